# baseline (speedup 1.0000x reference)
import functools

import jax
import jax.numpy as jnp
from jax import lax
from jax.experimental import pallas as pl
from jax.experimental.pallas import tpu as pltpu
from jax.experimental.pallas import tpu_sc as plsc

NC = 2
NS = 16
NW = NC * NS
LANES = 16
EPSILON = 1e-6
NACC = 8


def _rsqrt(x):
    i = lax.bitcast_convert_type(x, jnp.int32)
    i = jnp.int32(0x5F3759DF) - (i >> 1)
    y = lax.bitcast_convert_type(i, jnp.float32)
    for _ in range(3):
        y = y * (jnp.float32(1.5) - jnp.float32(0.5) * x * y * y)
    return y


def _tree_sum(vals):
    vals = list(vals)
    while len(vals) > 1:
        nxt = [a + b for a, b in zip(vals[0::2], vals[1::2])]
        if len(vals) % 2:
            nxt.append(vals[-1])
        vals = nxt
    return vals[0]


def kernel(input_ids, token_table, pos_table, ln_scale, ln_bias):
    B, S = input_ids.shape
    V, H = token_table.shape
    SP = S // NW
    NJ = H // LANES

    assert S % NW == 0 and H % LANES == 0 and SP == LANES and B % 2 == 0

    ids_w = (input_ids.astype(jnp.int32)
             .reshape(B, NW, SP).transpose(1, 0, 2).reshape(NW, B * SP))

    mesh = plsc.VectorSubcoreMesh(core_axis_name="c", subcore_axis_name="s")

    @functools.partial(
        pl.kernel,
        mesh=mesh,
        out_type=jax.ShapeDtypeStruct((B, S, H), jnp.float32),
        compiler_params=pltpu.CompilerParams(needs_layout_passes=False),
        scratch_types=[
            pltpu.VMEM((B * SP,), jnp.int32),
            pltpu.VMEM((SP, H), jnp.float32),
            pltpu.VMEM((H,), jnp.float32),
            pltpu.VMEM((H,), jnp.float32),
            pltpu.VMEM((2 * SP, H), jnp.float32),
            pltpu.VMEM((2 * SP, H), jnp.float32),
            pltpu.VMEM((2 * SP, H), jnp.float32),
            pltpu.VMEM((2 * SP, H), jnp.float32),
            pltpu.SMEM((2, LANES), jnp.float32),
            pltpu.SemaphoreType.DMA,
            pltpu.SemaphoreType.DMA,
            pltpu.SemaphoreType.DMA,
            pltpu.SemaphoreType.DMA,
        ],
    )
    def emb_kernel(ids_hbm, tok_hbm, pos_hbm, scale_hbm, bias_hbm, out_hbm,
                   idx_v, pos_v, scale_v, bias_v, in0, in1, ou0, ou1, stat_v,
                   gi0, gi1, go0, go1):
        wid = lax.axis_index("s") * NC + lax.axis_index("c")
        s0 = wid * SP

        pltpu.sync_copy(ids_hbm.at[wid], idx_v)
        cp_pos = pltpu.async_copy(pos_hbm.at[pl.ds(s0, SP), :], pos_v, go0)
        cp_sc = pltpu.async_copy(scale_hbm, scale_v, go0)
        cp_bi = pltpu.async_copy(bias_hbm, bias_v, go0)

        inv_h = jnp.float32(1.0 / H)
        ins, outs = (in0, in1), (ou0, ou1)
        gis, gos = (gi0, gi1), (go0, go1)

        def gather_start(c, buf, sem):
            pltpu.async_copy(
                tok_hbm.at[idx_v.at[pl.ds(c * 2 * SP, 2 * SP)]], buf, sem)

        def gather_wait(c, buf, sem):
            pltpu.make_async_copy(
                tok_hbm.at[idx_v.at[pl.ds(c * 2 * SP, 2 * SP)]], buf,
                sem).wait()

        def write_start(c, buf, sem):
            for g in range(2):
                pltpu.async_copy(
                    buf.at[pl.ds(g * SP, SP)],
                    out_hbm.at[2 * c + g, pl.ds(s0, SP), :], sem)

        def write_wait(c, buf, sem):
            for g in range(2):
                pltpu.make_async_copy(
                    buf.at[pl.ds(g * SP, SP)],
                    out_hbm.at[2 * c + g, pl.ds(s0, SP), :], sem).wait()

        def compute(src, dst, off):
            def one_row(r, c):
                accs = []
                accq = []
                for j in range(NJ):
                    sl = pl.ds(j * LANES, LANES)
                    x = src[off + r, sl] + pos_v[r, sl]
                    dst[off + r, sl] = x
                    if j < NACC:
                        accs.append(x)
                        accq.append(x * x)
                    else:
                        k = j % NACC
                        accs[k] = accs[k] + x
                        accq[k] = accq[k] + x * x
                mean = jnp.sum(_tree_sum(accs)) * inv_h
                var = jnp.sum(_tree_sum(accq)) * inv_h - mean * mean
                rstd = _rsqrt(var + jnp.float32(EPSILON))
                stat_v[0, r] = rstd
                stat_v[1, r] = -(mean * rstd)
                return c

            lax.fori_loop(0, SP, one_row, 0)

            a_s = [stat_v[0, r] for r in range(SP)]
            b_s = [stat_v[1, r] for r in range(SP)]

            def colblk(j, c):
                sl = pl.ds(j * LANES, LANES)
                sc = scale_v[sl]
                bi = bias_v[sl]
                for r in range(SP):
                    x = dst[off + r, sl]
                    dst[off + r, sl] = (x * a_s[r] + b_s[r]) * sc + bi
                return c

            lax.fori_loop(0, NJ, colblk, 0)

        NCH = B // 2
        gather_start(0, in0, gi0)
        gather_start(1, in1, gi1)
        cp_pos.wait()
        cp_sc.wait()
        cp_bi.wait()

        def pair(i, carry):
            for p in range(2):
                c = 2 * i + p
                gather_wait(c, ins[p], gis[p])

                @pl.when(i >= 1)
                def _():
                    write_wait(c - 2, outs[p], gos[p])

                compute(ins[p], outs[p], 0)
                compute(ins[p], outs[p], SP)

                @pl.when(i < (NCH // 2 - 1))
                def _():
                    gather_start(c + 2, ins[p], gis[p])

                write_start(c, outs[p], gos[p])
            return carry

        lax.fori_loop(0, NCH // 2, pair, 0)
        write_wait(NCH - 2, ou0, go0)
        write_wait(NCH - 1, ou1, go1)

    return emb_kernel(ids_w, token_table, pos_table, ln_scale, ln_bias)


# speedup vs baseline: 1.5459x; 1.5459x over previous
import functools

import jax
import jax.numpy as jnp
from jax import lax
from jax.experimental import pallas as pl
from jax.experimental.pallas import tpu as pltpu
from jax.experimental.pallas import tpu_sc as plsc

NC = 2
NS = 16
NW = NC * NS
LANES = 16
EPSILON = 1e-6
NACC = 8


def _rsqrt(x):
    i = lax.bitcast_convert_type(x, jnp.int32)
    i = jnp.int32(0x5F3759DF) - (i >> 1)
    y = lax.bitcast_convert_type(i, jnp.float32)
    for _ in range(3):
        y = y * (jnp.float32(1.5) - jnp.float32(0.5) * x * y * y)
    return y


def _tree_sum(vals):
    vals = list(vals)
    while len(vals) > 1:
        nxt = [a + b for a, b in zip(vals[0::2], vals[1::2])]
        if len(vals) % 2:
            nxt.append(vals[-1])
        vals = nxt
    return vals[0]


def kernel(input_ids, token_table, pos_table, ln_scale, ln_bias):
    B, S = input_ids.shape
    V, H = token_table.shape
    SP = S // NW
    NJ = H // LANES

    assert S % NW == 0 and H % LANES == 0 and SP == LANES and B % 2 == 0

    ids_w = (input_ids.astype(jnp.int32)
             .reshape(B, NW, SP).transpose(1, 0, 2).reshape(NW, B * SP))

    mesh = plsc.VectorSubcoreMesh(core_axis_name="c", subcore_axis_name="s")

    @functools.partial(
        pl.kernel,
        mesh=mesh,
        out_type=jax.ShapeDtypeStruct((B, S, H), jnp.float32),
        compiler_params=pltpu.CompilerParams(needs_layout_passes=False),
        scratch_types=[
            pltpu.VMEM((B * SP,), jnp.int32),
            pltpu.VMEM((SP, H), jnp.float32),
            pltpu.VMEM((H,), jnp.float32),
            pltpu.VMEM((H,), jnp.float32),
            pltpu.VMEM((SP, H), jnp.float32),
            pltpu.VMEM((SP, H), jnp.float32),
            pltpu.VMEM((SP, H), jnp.float32),
            pltpu.VMEM((SP, H), jnp.float32),
            pltpu.SMEM((2, LANES), jnp.float32),
            pltpu.SemaphoreType.DMA,
            pltpu.SemaphoreType.DMA,
            pltpu.SemaphoreType.DMA,
            pltpu.SemaphoreType.DMA,
        ],
    )
    def emb_kernel(ids_hbm, tok_hbm, pos_hbm, scale_hbm, bias_hbm, out_hbm,
                   idx_v, pos_v, scale_v, bias_v, in0, in1, ou0, ou1, stat_v,
                   gi0, gi1, go0, go1):
        wid = lax.axis_index("s") * NC + lax.axis_index("c")
        s0 = wid * SP

        pltpu.sync_copy(ids_hbm.at[wid], idx_v)
        cp_pos = pltpu.async_copy(pos_hbm.at[pl.ds(s0, SP), :], pos_v, go0)
        cp_sc = pltpu.async_copy(scale_hbm, scale_v, go0)
        cp_bi = pltpu.async_copy(bias_hbm, bias_v, go0)

        inv_h = jnp.float32(1.0 / H)
        ins, outs = (in0, in1), (ou0, ou1)
        gis, gos = (gi0, gi1), (go0, go1)

        def gather_start(b, buf, sem):
            pltpu.async_copy(tok_hbm.at[idx_v.at[pl.ds(b * SP, SP)]], buf, sem)

        def gather_wait(b, buf, sem):
            pltpu.make_async_copy(
                tok_hbm.at[idx_v.at[pl.ds(b * SP, SP)]], buf, sem).wait()

        def write_start(b, buf, sem):
            pltpu.async_copy(buf, out_hbm.at[b, pl.ds(s0, SP), :], sem)

        def write_wait(b, buf, sem):
            pltpu.make_async_copy(
                buf, out_hbm.at[b, pl.ds(s0, SP), :], sem).wait()

        def compute(src, dst):
            @plsc.parallel_loop(0, SP)
            def one_row(r):
                accs = []
                accq = []
                for j in range(NJ):
                    sl = pl.ds(j * LANES, LANES)
                    x = src[r, sl] + pos_v[r, sl]
                    dst[r, sl] = x
                    if j < NACC:
                        accs.append(x)
                        accq.append(x * x)
                    else:
                        k = j % NACC
                        accs[k] = accs[k] + x
                        accq[k] = accq[k] + x * x
                mean = jnp.sum(_tree_sum(accs)) * inv_h
                var = jnp.sum(_tree_sum(accq)) * inv_h - mean * mean
                rstd = _rsqrt(var + jnp.float32(EPSILON))
                stat_v[0, r] = rstd
                stat_v[1, r] = -(mean * rstd)

            a_s = [stat_v[0, r] for r in range(SP)]
            b_s = [stat_v[1, r] for r in range(SP)]

            @plsc.parallel_loop(0, NJ)
            def colblk(j):
                sl = pl.ds(j * LANES, LANES)
                sc = scale_v[sl]
                bi = bias_v[sl]
                for r in range(SP):
                    x = dst[r, sl]
                    dst[r, sl] = (x * a_s[r] + b_s[r]) * sc + bi

        gather_start(0, in0, gi0)
        gather_start(1, in1, gi1)
        cp_pos.wait()
        cp_sc.wait()
        cp_bi.wait()

        def pair(i, carry):
            for p in range(2):
                b = 2 * i + p
                gather_wait(b, ins[p], gis[p])

                @pl.when(i >= 1)
                def _():
                    write_wait(b - 2, outs[p], gos[p])

                compute(ins[p], outs[p])

                @pl.when(i < (B // 2 - 1))
                def _():
                    gather_start(b + 2, ins[p], gis[p])

                write_start(b, outs[p], gos[p])
            return carry

        lax.fori_loop(0, B // 2, pair, 0)
        write_wait(B - 2, ou0, go0)
        write_wait(B - 1, ou1, go1)

    return emb_kernel(ids_w, token_table, pos_table, ln_scale, ln_bias)


# speedup vs baseline: 1.9433x; 1.2571x over previous
import functools

import jax
import jax.numpy as jnp
from jax import lax
from jax.experimental import pallas as pl
from jax.experimental.pallas import tpu as pltpu
from jax.experimental.pallas import tpu_sc as plsc

NC = 2
NS = 16
NW = NC * NS
LANES = 16
EPSILON = 1e-6
NACC = 4


def _rsqrt(x):
    i = lax.bitcast_convert_type(x, jnp.int32)
    i = jnp.int32(0x5F3759DF) - (i >> 1)
    y = lax.bitcast_convert_type(i, jnp.float32)
    for _ in range(3):
        y = y * (jnp.float32(1.5) - jnp.float32(0.5) * x * y * y)
    return y


def _tree_sum(vals):
    vals = list(vals)
    while len(vals) > 1:
        nxt = [a + b for a, b in zip(vals[0::2], vals[1::2])]
        if len(vals) % 2:
            nxt.append(vals[-1])
        vals = nxt
    return vals[0]


def kernel(input_ids, token_table, pos_table, ln_scale, ln_bias):
    B, S = input_ids.shape
    V, H = token_table.shape
    SP = S // NW
    NJ = H // LANES

    assert S % NW == 0 and H % LANES == 0 and SP == LANES and B % 2 == 0

    ids_w = (input_ids.astype(jnp.int32)
             .reshape(B, NW, SP).transpose(1, 0, 2).reshape(NW, B * SP))

    mesh = plsc.VectorSubcoreMesh(core_axis_name="c", subcore_axis_name="s")

    @functools.partial(
        pl.kernel,
        mesh=mesh,
        out_type=jax.ShapeDtypeStruct((B, S, H), jnp.float32),
        compiler_params=pltpu.CompilerParams(needs_layout_passes=False),
        scratch_types=[
            pltpu.VMEM((B * SP,), jnp.int32),
            pltpu.VMEM((SP, H), jnp.float32),
            pltpu.VMEM((H,), jnp.float32),
            pltpu.VMEM((H,), jnp.float32),
            pltpu.VMEM((SP, H), jnp.float32),
            pltpu.VMEM((SP, H), jnp.float32),
            pltpu.VMEM((SP, H), jnp.float32),
            pltpu.VMEM((SP, H), jnp.float32),
            pltpu.SMEM((2, LANES), jnp.float32),
            pltpu.SemaphoreType.DMA,
            pltpu.SemaphoreType.DMA,
            pltpu.SemaphoreType.DMA,
            pltpu.SemaphoreType.DMA,
        ],
    )
    def emb_kernel(ids_hbm, tok_hbm, pos_hbm, scale_hbm, bias_hbm, out_hbm,
                   idx_v, pos_v, scale_v, bias_v, in0, in1, ou0, ou1, stat_v,
                   gi0, gi1, go0, go1):
        wid = lax.axis_index("s") * NC + lax.axis_index("c")
        s0 = wid * SP

        pltpu.sync_copy(ids_hbm.at[wid], idx_v)
        cp_pos = pltpu.async_copy(pos_hbm.at[pl.ds(s0, SP), :], pos_v, go0)
        cp_sc = pltpu.async_copy(scale_hbm, scale_v, go0)
        cp_bi = pltpu.async_copy(bias_hbm, bias_v, go0)

        inv_h = jnp.float32(1.0 / H)
        ins, outs = (in0, in1), (ou0, ou1)
        gis, gos = (gi0, gi1), (go0, go1)

        def gather_start(b, buf, sem):
            pltpu.async_copy(tok_hbm.at[idx_v.at[pl.ds(b * SP, SP)]], buf, sem)

        def gather_wait(b, buf, sem):
            pltpu.make_async_copy(
                tok_hbm.at[idx_v.at[pl.ds(b * SP, SP)]], buf, sem).wait()

        def write_start(b, buf, sem):
            pltpu.async_copy(buf, out_hbm.at[b, pl.ds(s0, SP), :], sem)

        def write_wait(b, buf, sem):
            pltpu.make_async_copy(
                buf, out_hbm.at[b, pl.ds(s0, SP), :], sem).wait()

        def compute(src, dst):
            @plsc.parallel_loop(0, SP)
            def one_row(r):
                accs = []
                accq = []
                for j in range(NJ):
                    sl = pl.ds(j * LANES, LANES)
                    x = src[r, sl] + pos_v[r, sl]
                    dst[r, sl] = x
                    if j < NACC:
                        accs.append(x)
                        accq.append(x * x)
                    else:
                        k = j % NACC
                        accs[k] = accs[k] + x
                        accq[k] = accq[k] + x * x
                mean = jnp.sum(_tree_sum(accs)) * inv_h
                var = jnp.sum(_tree_sum(accq)) * inv_h - mean * mean
                rstd = _rsqrt(var + jnp.float32(EPSILON))
                stat_v[0, r] = rstd
                stat_v[1, r] = -(mean * rstd)

            a_s = [stat_v[0, r] for r in range(SP)]
            b_s = [stat_v[1, r] for r in range(SP)]

            @plsc.parallel_loop(0, NJ)
            def colblk(j):
                sl = pl.ds(j * LANES, LANES)
                sc = scale_v[sl]
                bi = bias_v[sl]
                for r in range(SP):
                    x = dst[r, sl]
                    dst[r, sl] = (x * a_s[r] + b_s[r]) * sc + bi

        gather_start(0, in0, gi0)
        gather_start(1, in1, gi1)
        cp_pos.wait()
        cp_sc.wait()
        cp_bi.wait()

        def pair(i, carry):
            for p in range(2):
                b = 2 * i + p
                gather_wait(b, ins[p], gis[p])

                @pl.when(i >= 1)
                def _():
                    write_wait(b - 2, outs[p], gos[p])

                compute(ins[p], outs[p])

                @pl.when(i < (B // 2 - 1))
                def _():
                    gather_start(b + 2, ins[p], gis[p])

                write_start(b, outs[p], gos[p])
            return carry

        lax.fori_loop(0, B // 2, pair, 0)
        write_wait(B - 2, ou0, go0)
        write_wait(B - 1, ou1, go1)

    return emb_kernel(ids_w, token_table, pos_table, ln_scale, ln_bias)


# speedup vs baseline: 1.9974x; 1.0278x over previous
import functools

import jax
import jax.numpy as jnp
from jax import lax
from jax.experimental import pallas as pl
from jax.experimental.pallas import tpu as pltpu
from jax.experimental.pallas import tpu_sc as plsc

NC = 2
NS = 16
NW = NC * NS
LANES = 16
EPSILON = 1e-6
NACC = 4


def _rsqrt(x):
    i = lax.bitcast_convert_type(x, jnp.int32)
    i = jnp.int32(0x5F3759DF) - (i >> 1)
    y = lax.bitcast_convert_type(i, jnp.float32)
    for _ in range(3):
        y = y * (jnp.float32(1.5) - jnp.float32(0.5) * x * y * y)
    return y


def _tree_sum(vals):
    vals = list(vals)
    while len(vals) > 1:
        nxt = [a + b for a, b in zip(vals[0::2], vals[1::2])]
        if len(vals) % 2:
            nxt.append(vals[-1])
        vals = nxt
    return vals[0]


def kernel(input_ids, token_table, pos_table, ln_scale, ln_bias):
    B, S = input_ids.shape
    V, H = token_table.shape
    SP = S // NW
    NJ = H // LANES

    assert S % NW == 0 and H % LANES == 0 and SP == LANES and B % 2 == 0

    ids_w = (input_ids.astype(jnp.int32)
             .reshape(B, NW, SP).transpose(1, 0, 2).reshape(NW, B * SP))

    mesh = plsc.VectorSubcoreMesh(core_axis_name="c", subcore_axis_name="s")

    @functools.partial(
        pl.kernel,
        mesh=mesh,
        out_type=jax.ShapeDtypeStruct((B, S, H), jnp.float32),
        compiler_params=pltpu.CompilerParams(needs_layout_passes=False),
        scratch_types=[
            pltpu.VMEM((B * SP,), jnp.int32),
            pltpu.VMEM((SP, H), jnp.float32),
            pltpu.VMEM((H,), jnp.float32),
            pltpu.VMEM((H,), jnp.float32),
            pltpu.VMEM((SP, H), jnp.float32),
            pltpu.VMEM((SP, H), jnp.float32),
            pltpu.VMEM((SP, H), jnp.float32),
            pltpu.VMEM((SP, H), jnp.float32),
            pltpu.SMEM((2, LANES), jnp.float32),
            pltpu.SemaphoreType.DMA,
            pltpu.SemaphoreType.DMA,
            pltpu.SemaphoreType.DMA,
            pltpu.SemaphoreType.DMA,
        ],
    )
    def emb_kernel(ids_hbm, tok_hbm, pos_hbm, scale_hbm, bias_hbm, out_hbm,
                   idx_v, pos_v, scale_v, bias_v, in0, in1, ou0, ou1, stat_v,
                   gi0, gi1, go0, go1):
        wid = lax.axis_index("s") * NC + lax.axis_index("c")
        s0 = wid * SP

        pltpu.sync_copy(ids_hbm.at[wid], idx_v)
        cp_pos = pltpu.async_copy(pos_hbm.at[pl.ds(s0, SP), :], pos_v, go0)
        cp_sc = pltpu.async_copy(scale_hbm, scale_v, go0)
        cp_bi = pltpu.async_copy(bias_hbm, bias_v, go0)

        inv_h = jnp.float32(1.0 / H)
        ins, outs = (in0, in1), (ou0, ou1)
        gis, gos = (gi0, gi1), (go0, go1)

        def gather_start(b, buf, sem):
            pltpu.async_copy(tok_hbm.at[idx_v.at[pl.ds(b * SP, SP)]], buf, sem)

        def gather_wait(b, buf, sem):
            pltpu.make_async_copy(
                tok_hbm.at[idx_v.at[pl.ds(b * SP, SP)]], buf, sem).wait()

        def write_start(b, buf, sem):
            pltpu.async_copy(buf, out_hbm.at[b, pl.ds(s0, SP), :], sem)

        def write_wait(b, buf, sem):
            pltpu.make_async_copy(
                buf, out_hbm.at[b, pl.ds(s0, SP), :], sem).wait()

        def pass1(src, dst):
            @plsc.parallel_loop(0, SP)
            def one_row(r):
                accs = []
                accq = []
                for j in range(NJ):
                    sl = pl.ds(j * LANES, LANES)
                    x = src[r, sl] + pos_v[r, sl]
                    dst[r, sl] = x
                    if j < NACC:
                        accs.append(x)
                        accq.append(x * x)
                    else:
                        k = j % NACC
                        accs[k] = accs[k] + x
                        accq[k] = accq[k] + x * x
                mean = jnp.sum(_tree_sum(accs)) * inv_h
                var = jnp.sum(_tree_sum(accq)) * inv_h - mean * mean
                rstd = _rsqrt(var + jnp.float32(EPSILON))
                stat_v[0, r] = rstd
                stat_v[1, r] = -(mean * rstd)

        def pass2(dst):
            a_s = [stat_v[0, r] for r in range(SP)]
            b_s = [stat_v[1, r] for r in range(SP)]

            @plsc.parallel_loop(0, NJ)
            def colblk(j):
                sl = pl.ds(j * LANES, LANES)
                sc = scale_v[sl]
                bi = bias_v[sl]
                for r in range(SP):
                    x = dst[r, sl]
                    dst[r, sl] = (x * a_s[r] + b_s[r]) * sc + bi

        gather_start(0, in0, gi0)
        gather_start(1, in1, gi1)
        cp_pos.wait()
        cp_sc.wait()
        cp_bi.wait()

        def pair(i, carry):
            for p in range(2):
                b = 2 * i + p
                gather_wait(b, ins[p], gis[p])

                @pl.when(i >= 1)
                def _():
                    write_wait(b - 2, outs[p], gos[p])

                pass1(ins[p], outs[p])

                @pl.when(i < (B // 2 - 1))
                def _():
                    gather_start(b + 2, ins[p], gis[p])

                pass2(outs[p])
                write_start(b, outs[p], gos[p])
            return carry

        lax.fori_loop(0, B // 2, pair, 0)
        write_wait(B - 2, ou0, go0)
        write_wait(B - 1, ou1, go1)

    return emb_kernel(ids_w, token_table, pos_table, ln_scale, ln_bias)
